# Initial kernel scaffold; baseline (speedup 1.0000x reference)
import jax
import jax.numpy as jnp
from jax.experimental import pallas as pl


def kernel(x, batch, y, W0, b0, W1, b1):
    raise NotImplementedError("write your pallas kernel here")


# speedup vs baseline: 4.8765x; 4.8765x over previous
import functools

import jax
import jax.numpy as jnp
from jax import lax
from jax.experimental import pallas as pl
from jax.experimental.pallas import tpu as pltpu
from jax.experimental.pallas import tpu_sc as plsc

N = 4096
D = 256
K = 32
RB = 256
NRB = N // RB
EB = 64
NEB = N // EB
BIGD = 1e30
USEDD = 1e38
BIGI = 2**30


def _topk_idx(x_ref, bc_ref, br_ref, i):
    xall = x_ref[...]
    xrb = x_ref[pl.ds(i * RB, RB), :]
    g = jax.lax.dot_general(xrb, xall, (((1,), (1,)), ((), ())),
                            preferred_element_type=jnp.float32)
    x2 = xall * xall
    ones_row = jnp.ones((1, D), jnp.float32)
    d2row = jax.lax.dot_general(ones_row, x2, (((1,), (1,)), ((), ())),
                                preferred_element_type=jnp.float32,
                                precision=lax.Precision.HIGHEST)
    d2col = jnp.sum(xrb * xrb, axis=1, keepdims=True)
    dist = d2col - 2.0 * g + d2row

    bad = bc_ref[...] != br_ref[...]
    vals0 = jnp.where(bad, BIGD, dist)

    col = lax.broadcasted_iota(jnp.int32, (RB, N), 1)
    lane = lax.broadcasted_iota(jnp.int32, (RB, K), 1)
    idx0 = jnp.zeros((RB, K), jnp.int32)

    def step(k, carry):
        vals, idxm = carry
        m = jnp.min(vals, axis=1, keepdims=True)
        cand = jnp.where(vals == m, col, BIGI)
        am = jnp.min(cand, axis=1, keepdims=True)
        idxm = jnp.where(lane == k, am, idxm)
        vals = jnp.where(col == am, USEDD, vals)
        return vals, idxm

    _, idxm = lax.fori_loop(0, K, step, (vals0, idx0))
    return idxm


def _l1_body(x_ref, bc_ref, br_ref, w_ref, idx_ref, a_ref):
    i = pl.program_id(0)
    idx_ref[...] = _topk_idx(x_ref, bc_ref, br_ref, i)
    xrb = x_ref[pl.ds(i * RB, RB), :]
    wt = w_ref[0:D, :]
    a_ref[...] = jax.lax.dot_general(xrb, wt, (((1,), (0,)), ((), ())),
                                     preferred_element_type=jnp.float32)


def _l1_call(x, bcol, brow, W):
    return pl.pallas_call(
        _l1_body,
        grid=(NRB,),
        in_specs=[
            pl.BlockSpec((N, D), lambda i: (0, 0)),
            pl.BlockSpec((RB, 1), lambda i: (i, 0)),
            pl.BlockSpec((1, N), lambda i: (0, 0)),
            pl.BlockSpec((2 * D, D), lambda i: (0, 0)),
        ],
        out_specs=[
            pl.BlockSpec((RB, K), lambda i: (i, 0)),
            pl.BlockSpec((RB, D), lambda i: (i, 0)),
        ],
        out_shape=[
            jax.ShapeDtypeStruct((N, K), jnp.int32),
            jax.ShapeDtypeStruct((N, D), jnp.float32),
        ],
    )(x, bcol, brow, W)


def _l2_body(x_ref, bc_ref, br_ref, w_ref, b_ref, idx_ref, a_ref, bm_ref):
    i = pl.program_id(0)
    idx_ref[...] = _topk_idx(x_ref, bc_ref, br_ref, i)
    xrb = x_ref[pl.ds(i * RB, RB), :]
    wt = w_ref[0:D, :]
    wb = w_ref[D : 2 * D, :]
    a_ref[...] = (
        jax.lax.dot_general(xrb, wt - wb, (((1,), (0,)), ((), ())),
                            preferred_element_type=jnp.float32,
                            precision=lax.Precision.HIGHEST)
        + b_ref[...]
    )
    bm_ref[...] = jax.lax.dot_general(xrb, wb, (((1,), (0,)), ((), ())),
                                      preferred_element_type=jnp.float32,
                                      precision=lax.Precision.HIGHEST)


def _l2_call(x, bcol, brow, W, brow_bias):
    return pl.pallas_call(
        _l2_body,
        grid=(NRB,),
        in_specs=[
            pl.BlockSpec((N, D), lambda i: (0, 0)),
            pl.BlockSpec((RB, 1), lambda i: (i, 0)),
            pl.BlockSpec((1, N), lambda i: (0, 0)),
            pl.BlockSpec((2 * D, D), lambda i: (0, 0)),
            pl.BlockSpec((1, D), lambda i: (0, 0)),
        ],
        out_specs=[
            pl.BlockSpec((RB, K), lambda i: (i, 0)),
            pl.BlockSpec((RB, D), lambda i: (i, 0)),
            pl.BlockSpec((RB, D), lambda i: (i, 0)),
        ],
        out_shape=[
            jax.ShapeDtypeStruct((N, K), jnp.int32),
            jax.ShapeDtypeStruct((N, D), jnp.float32),
            jax.ShapeDtypeStruct((N, D), jnp.float32),
        ],
    )(x, bcol, brow, W, brow_bias)


_NC = 2
_NS = 16
_L = 16
_NW = _NC * _NS
_RPW = N // _NW


def _mesh():
    return plsc.VectorSubcoreMesh(core_axis_name="c", subcore_axis_name="s",
                                  num_cores=_NC, num_subcores=_NS)


def _wid_base():
    wid = lax.axis_index("s") * _NC + lax.axis_index("c")
    return wid * _RPW


def _sc_gather_body(idx_hbm, x_hbm, out_hbm, idx_v, rows_v, sem):
    base = _wid_base()
    pltpu.sync_copy(idx_hbm.at[pl.ds(base, _RPW)], idx_v)

    def row(r, carry):
        pltpu.async_copy(x_hbm.at[idx_v.at[r]], rows_v, sem).wait()
        pltpu.sync_copy(rows_v, out_hbm.at[pl.ds((base + r) * K, K)])
        return carry

    lax.fori_loop(0, _RPW, row, 0)


@functools.cache
def _sc_gather_call():
    return pl.kernel(
        _sc_gather_body,
        out_type=jax.ShapeDtypeStruct((N * K, D), jnp.float32),
        mesh=_mesh(),
        scratch_types=[
            pltpu.VMEM((_RPW, K), jnp.int32),
            pltpu.VMEM((K, D), jnp.float32),
            pltpu.SemaphoreType.DMA,
        ],
    )


def _sc_gather_max_body(idx_hbm, bm_hbm, a_hbm, out_hbm,
                        idx_v, a_v, h_v, rows_v, sem):
    base = _wid_base()
    pltpu.sync_copy(idx_hbm.at[pl.ds(base, _RPW)], idx_v)
    pltpu.sync_copy(a_hbm.at[pl.ds(base, _RPW)], a_v)

    def row(r, carry):
        pltpu.async_copy(bm_hbm.at[idx_v.at[r]], rows_v, sem).wait()
        for g in range(D // _L):
            sl = pl.ds(g * _L, _L)
            acc = rows_v[0, sl]
            for j in range(1, K):
                acc = jnp.maximum(acc, rows_v[j, sl])
            h_v[r, sl] = jnp.maximum(acc + a_v[r, sl], 0.0)
        return carry

    lax.fori_loop(0, _RPW, row, 0)
    pltpu.sync_copy(h_v, out_hbm.at[pl.ds(base, _RPW)])


@functools.cache
def _sc_gather_max_call():
    return pl.kernel(
        _sc_gather_max_body,
        out_type=jax.ShapeDtypeStruct((N, D), jnp.float32),
        mesh=_mesh(),
        scratch_types=[
            pltpu.VMEM((_RPW, K), jnp.int32),
            pltpu.VMEM((_RPW, D), jnp.float32),
            pltpu.VMEM((_RPW, D), jnp.float32),
            pltpu.VMEM((K, D), jnp.float32),
            pltpu.SemaphoreType.DMA,
        ],
    )


def _edge_body(xj_ref, x_ref, w_ref, b_ref, a_ref, h_ref):
    xj3 = xj_ref[...].reshape(EB, K, D)
    xi = x_ref[...]
    diff = (xj3 - xi[:, None, :]).astype(jnp.bfloat16).reshape(EB * K, D)
    wb = w_ref[D : 2 * D, :].astype(jnp.bfloat16)
    e = jax.lax.dot_general(diff, wb, (((1,), (0,)), ((), ())),
                            preferred_element_type=jnp.float32)
    p = e.reshape(EB, K, D) + a_ref[...][:, None, :] + b_ref[...][None, :, :]
    h_ref[...] = jnp.max(jnp.maximum(p, 0.0), axis=1)


def _edge_call(xj, x, W, b, a):
    return pl.pallas_call(
        _edge_body,
        grid=(NEB,),
        in_specs=[
            pl.BlockSpec((EB * K, D), lambda i: (i, 0)),
            pl.BlockSpec((EB, D), lambda i: (i, 0)),
            pl.BlockSpec((2 * D, D), lambda i: (0, 0)),
            pl.BlockSpec((1, D), lambda i: (0, 0)),
            pl.BlockSpec((EB, D), lambda i: (i, 0)),
        ],
        out_specs=pl.BlockSpec((EB, D), lambda i: (i, 0)),
        out_shape=jax.ShapeDtypeStruct((N, D), jnp.float32),
    )(xj, x, W, b, a)


def _loss_body(h_ref, y_ref, o_ref):
    i = pl.program_id(0)

    @pl.when(i == 0)
    def _():
        o_ref[...] = jnp.zeros_like(o_ref)

    d = h_ref[...] - y_ref[...]
    o_ref[...] += jnp.sum(d * d).reshape(1, 1) / jnp.float32(N * D)


def _tc_loss(h, y):
    return pl.pallas_call(
        _loss_body,
        grid=(NRB,),
        in_specs=[
            pl.BlockSpec((RB, D), lambda i: (i, 0)),
            pl.BlockSpec((RB, D), lambda i: (i, 0)),
        ],
        out_specs=pl.BlockSpec((1, 1), lambda i: (0, 0)),
        out_shape=jax.ShapeDtypeStruct((1, 1), jnp.float32),
    )(h, y)


def kernel(x, batch, y, W0, b0, W1, b1):
    bf = batch.astype(jnp.float32)
    bcol = bf.reshape(N, 1)
    brow = bf.reshape(1, N)

    idx0, a1 = _l1_call(x, bcol, brow, W0)
    xj = _sc_gather_call()(idx0, x)
    h1 = _edge_call(xj, x, W0, b0.reshape(1, D), a1)

    idx1, a2, bm2 = _l2_call(h1, bcol, brow, W1, b1.reshape(1, D))
    h2 = _sc_gather_max_call()(idx1, bm2, a2)

    loss = _tc_loss(h2, y)[0, 0]
    return (h2, loss)


# speedup vs baseline: 8.8548x; 1.8158x over previous
import functools

import jax
import jax.numpy as jnp
from jax import lax
from jax.experimental import pallas as pl
from jax.experimental.pallas import tpu as pltpu
from jax.experimental.pallas import tpu_sc as plsc

N = 4096
D = 256
K = 32
RB = 256
NRB = N // RB
EB = 64
NEB = N // EB
BIGD = 1e30
USEDD = 1e38
BIGI = 2**30


NCH = N // 128
Q = 6
SENT = 3.0e38


def _topk_slow(vals0):
    col = lax.broadcasted_iota(jnp.int32, (RB, N), 1)
    lane = lax.broadcasted_iota(jnp.int32, (RB, K), 1)
    idx0 = jnp.zeros((RB, K), jnp.int32)

    def step(k, carry):
        vals, idxm = carry
        m = jnp.min(vals, axis=1, keepdims=True)
        cand = jnp.where(vals == m, col, BIGI)
        am = jnp.min(cand, axis=1, keepdims=True)
        idxm = jnp.where(lane == k, am, idxm)
        vals = jnp.where(col == am, USEDD, vals)
        return vals, idxm

    _, idxm = lax.fori_loop(0, K, step, (vals0, idx0))
    return idxm


def _topk_idx(x_ref, bc_ref, br_ref, i, idx_ref):
    xall = x_ref[...]
    xrb = x_ref[pl.ds(i * RB, RB), :]
    g = jax.lax.dot_general(xrb, xall, (((1,), (1,)), ((), ())),
                            preferred_element_type=jnp.float32)
    x2 = xall * xall
    ones_row = jnp.ones((1, D), jnp.float32)
    d2row = jax.lax.dot_general(ones_row, x2, (((1,), (1,)), ((), ())),
                                preferred_element_type=jnp.float32,
                                precision=lax.Precision.HIGHEST)
    d2col = jnp.sum(xrb * xrb, axis=1, keepdims=True)
    dist = d2col - 2.0 * g + d2row

    bad = bc_ref[...] != br_ref[...]
    vals0 = jnp.where(bad, BIGD, dist)

    l128 = lax.broadcasted_iota(jnp.int32, (RB, 128), 1)
    mq = [jnp.full((RB, 128), SENT, jnp.float32) for _ in range(Q)]
    jq = [jnp.zeros((RB, 128), jnp.int32) for _ in range(Q)]
    for c in range(NCH):
        v = vals0[:, c * 128 : (c + 1) * 128]
        jfill = l128 + (c * 128)
        b = [v < mq[t] for t in range(Q)]
        for t in range(Q - 1, 0, -1):
            mq[t] = jnp.where(b[t], jnp.where(b[t - 1], mq[t - 1], v), mq[t])
            jq[t] = jnp.where(b[t], jnp.where(b[t - 1], jq[t - 1], jfill), jq[t])
        mq[0] = jnp.where(b[0], v, mq[0])
        jq[0] = jnp.where(b[0], jfill, jq[0])

    lane = lax.broadcasted_iota(jnp.int32, (RB, K), 1)
    idxm = jnp.zeros((RB, K), jnp.int32)
    pops = jnp.zeros((RB, 128), jnp.int32)

    def pstep(k, carry):
        mq, jq, idxm, pops = carry
        mq = list(mq)
        jq = list(jq)
        m = jnp.min(mq[0], axis=1, keepdims=True)
        jc = jnp.where(mq[0] == m, jq[0], BIGI)
        am = jnp.min(jc, axis=1, keepdims=True)
        idxm = jnp.where(lane == k, am, idxm)
        popm = jc == am
        for t in range(Q - 1):
            mq[t] = jnp.where(popm, mq[t + 1], mq[t])
            jq[t] = jnp.where(popm, jq[t + 1], jq[t])
        mq[Q - 1] = jnp.where(popm, SENT, mq[Q - 1])
        pops = pops + popm.astype(jnp.int32)
        return tuple(mq), tuple(jq), idxm, pops

    (_, _, idxm, pops) = lax.fori_loop(
        0, K, pstep, (tuple(mq), tuple(jq), idxm, pops))
    idx_ref[...] = idxm

    overflow = jnp.any(pops >= Q)

    @pl.when(overflow)
    def _():
        idx_ref[...] = _topk_slow(vals0)


def _l1_body(x_ref, bc_ref, br_ref, w_ref, idx_ref, a_ref):
    i = pl.program_id(0)
    _topk_idx(x_ref, bc_ref, br_ref, i, idx_ref)
    xrb = x_ref[pl.ds(i * RB, RB), :]
    wt = w_ref[0:D, :]
    a_ref[...] = jax.lax.dot_general(xrb, wt, (((1,), (0,)), ((), ())),
                                     preferred_element_type=jnp.float32)


def _l1_call(x, bcol, brow, W):
    return pl.pallas_call(
        _l1_body,
        grid=(NRB,),
        in_specs=[
            pl.BlockSpec((N, D), lambda i: (0, 0)),
            pl.BlockSpec((RB, 1), lambda i: (i, 0)),
            pl.BlockSpec((1, N), lambda i: (0, 0)),
            pl.BlockSpec((2 * D, D), lambda i: (0, 0)),
        ],
        out_specs=[
            pl.BlockSpec((RB, K), lambda i: (i, 0)),
            pl.BlockSpec((RB, D), lambda i: (i, 0)),
        ],
        out_shape=[
            jax.ShapeDtypeStruct((N, K), jnp.int32),
            jax.ShapeDtypeStruct((N, D), jnp.float32),
        ],
    )(x, bcol, brow, W)


def _l2_body(x_ref, bc_ref, br_ref, w_ref, b_ref, idx_ref, a_ref, bm_ref):
    i = pl.program_id(0)
    _topk_idx(x_ref, bc_ref, br_ref, i, idx_ref)
    xrb = x_ref[pl.ds(i * RB, RB), :]
    wt = w_ref[0:D, :]
    wb = w_ref[D : 2 * D, :]
    a_ref[...] = (
        jax.lax.dot_general(xrb, wt - wb, (((1,), (0,)), ((), ())),
                            preferred_element_type=jnp.float32,
                            precision=lax.Precision.HIGHEST)
        + b_ref[...]
    )
    bm_ref[...] = jax.lax.dot_general(xrb, wb, (((1,), (0,)), ((), ())),
                                      preferred_element_type=jnp.float32,
                                      precision=lax.Precision.HIGHEST)


def _l2_call(x, bcol, brow, W, brow_bias):
    return pl.pallas_call(
        _l2_body,
        grid=(NRB,),
        in_specs=[
            pl.BlockSpec((N, D), lambda i: (0, 0)),
            pl.BlockSpec((RB, 1), lambda i: (i, 0)),
            pl.BlockSpec((1, N), lambda i: (0, 0)),
            pl.BlockSpec((2 * D, D), lambda i: (0, 0)),
            pl.BlockSpec((1, D), lambda i: (0, 0)),
        ],
        out_specs=[
            pl.BlockSpec((RB, K), lambda i: (i, 0)),
            pl.BlockSpec((RB, D), lambda i: (i, 0)),
            pl.BlockSpec((RB, D), lambda i: (i, 0)),
        ],
        out_shape=[
            jax.ShapeDtypeStruct((N, K), jnp.int32),
            jax.ShapeDtypeStruct((N, D), jnp.float32),
            jax.ShapeDtypeStruct((N, D), jnp.float32),
        ],
    )(x, bcol, brow, W, brow_bias)


_NC = 2
_NS = 16
_L = 16
_NW = _NC * _NS
_RPW = N // _NW


def _mesh():
    return plsc.VectorSubcoreMesh(core_axis_name="c", subcore_axis_name="s",
                                  num_cores=_NC, num_subcores=_NS)


def _wid_base():
    wid = lax.axis_index("s") * _NC + lax.axis_index("c")
    return wid * _RPW


def _sc_gather_body(idx_hbm, x_hbm, out_hbm, idx_v, rows_v, sem):
    base = _wid_base()
    pltpu.sync_copy(idx_hbm.at[pl.ds(base, _RPW)], idx_v)

    def row(r, carry):
        pltpu.async_copy(x_hbm.at[idx_v.at[r]], rows_v, sem).wait()
        pltpu.sync_copy(rows_v, out_hbm.at[pl.ds((base + r) * K, K)])
        return carry

    lax.fori_loop(0, _RPW, row, 0)


@functools.cache
def _sc_gather_call():
    return pl.kernel(
        _sc_gather_body,
        out_type=jax.ShapeDtypeStruct((N * K, D), jnp.float32),
        mesh=_mesh(),
        scratch_types=[
            pltpu.VMEM((_RPW, K), jnp.int32),
            pltpu.VMEM((K, D), jnp.float32),
            pltpu.SemaphoreType.DMA,
        ],
    )


def _sc_gather_max_body(idx_hbm, bm_hbm, a_hbm, out_hbm,
                        idx_v, a_v, h_v, rows_v, sem):
    base = _wid_base()
    pltpu.sync_copy(idx_hbm.at[pl.ds(base, _RPW)], idx_v)
    pltpu.sync_copy(a_hbm.at[pl.ds(base, _RPW)], a_v)

    def row(r, carry):
        pltpu.async_copy(bm_hbm.at[idx_v.at[r]], rows_v, sem).wait()
        for g in range(D // _L):
            sl = pl.ds(g * _L, _L)
            acc = rows_v[0, sl]
            for j in range(1, K):
                acc = jnp.maximum(acc, rows_v[j, sl])
            h_v[r, sl] = jnp.maximum(acc + a_v[r, sl], 0.0)
        return carry

    lax.fori_loop(0, _RPW, row, 0)
    pltpu.sync_copy(h_v, out_hbm.at[pl.ds(base, _RPW)])


@functools.cache
def _sc_gather_max_call():
    return pl.kernel(
        _sc_gather_max_body,
        out_type=jax.ShapeDtypeStruct((N, D), jnp.float32),
        mesh=_mesh(),
        scratch_types=[
            pltpu.VMEM((_RPW, K), jnp.int32),
            pltpu.VMEM((_RPW, D), jnp.float32),
            pltpu.VMEM((_RPW, D), jnp.float32),
            pltpu.VMEM((K, D), jnp.float32),
            pltpu.SemaphoreType.DMA,
        ],
    )


def _edge_body(xj_ref, x_ref, w_ref, b_ref, a_ref, h_ref):
    xj3 = xj_ref[...].reshape(EB, K, D)
    xi = x_ref[...]
    diff = (xj3 - xi[:, None, :]).astype(jnp.bfloat16).reshape(EB * K, D)
    wb = w_ref[D : 2 * D, :].astype(jnp.bfloat16)
    e = jax.lax.dot_general(diff, wb, (((1,), (0,)), ((), ())),
                            preferred_element_type=jnp.float32)
    p = e.reshape(EB, K, D) + a_ref[...][:, None, :] + b_ref[...][None, :, :]
    h_ref[...] = jnp.max(jnp.maximum(p, 0.0), axis=1)


def _edge_call(xj, x, W, b, a):
    return pl.pallas_call(
        _edge_body,
        grid=(NEB,),
        in_specs=[
            pl.BlockSpec((EB * K, D), lambda i: (i, 0)),
            pl.BlockSpec((EB, D), lambda i: (i, 0)),
            pl.BlockSpec((2 * D, D), lambda i: (0, 0)),
            pl.BlockSpec((1, D), lambda i: (0, 0)),
            pl.BlockSpec((EB, D), lambda i: (i, 0)),
        ],
        out_specs=pl.BlockSpec((EB, D), lambda i: (i, 0)),
        out_shape=jax.ShapeDtypeStruct((N, D), jnp.float32),
    )(xj, x, W, b, a)


def _loss_body(h_ref, y_ref, o_ref):
    i = pl.program_id(0)

    @pl.when(i == 0)
    def _():
        o_ref[...] = jnp.zeros_like(o_ref)

    d = h_ref[...] - y_ref[...]
    o_ref[...] += jnp.sum(d * d).reshape(1, 1) / jnp.float32(N * D)


def _tc_loss(h, y):
    return pl.pallas_call(
        _loss_body,
        grid=(NRB,),
        in_specs=[
            pl.BlockSpec((RB, D), lambda i: (i, 0)),
            pl.BlockSpec((RB, D), lambda i: (i, 0)),
        ],
        out_specs=pl.BlockSpec((1, 1), lambda i: (0, 0)),
        out_shape=jax.ShapeDtypeStruct((1, 1), jnp.float32),
    )(h, y)


def kernel(x, batch, y, W0, b0, W1, b1):
    bf = batch.astype(jnp.float32)
    bcol = bf.reshape(N, 1)
    brow = bf.reshape(1, N)

    idx0, a1 = _l1_call(x, bcol, brow, W0)
    xj = _sc_gather_call()(idx0, x)
    h1 = _edge_call(xj, x, W0, b0.reshape(1, D), a1)

    idx1, a2, bm2 = _l2_call(h1, bcol, brow, W1, b1.reshape(1, D))
    h2 = _sc_gather_max_call()(idx1, bm2, a2)

    loss = _tc_loss(h2, y)[0, 0]
    return (h2, loss)


# speedup vs baseline: 10.0039x; 1.1298x over previous
import functools

import jax
import jax.numpy as jnp
from jax import lax
from jax.experimental import pallas as pl
from jax.experimental.pallas import tpu as pltpu
from jax.experimental.pallas import tpu_sc as plsc

N = 4096
D = 256
K = 32
RB = 256
NRB = N // RB
EB = 64
NEB = N // EB
BIGD = 1e30
USEDD = 1e38
BIGI = 2**30


NCH = N // 128
Q = 6
SENT = 3.0e38


def _topk_slow(vals0):
    col = lax.broadcasted_iota(jnp.int32, (RB, N), 1)
    lane = lax.broadcasted_iota(jnp.int32, (RB, K), 1)
    idx0 = jnp.zeros((RB, K), jnp.int32)

    def step(k, carry):
        vals, idxm = carry
        m = jnp.min(vals, axis=1, keepdims=True)
        cand = jnp.where(vals == m, col, BIGI)
        am = jnp.min(cand, axis=1, keepdims=True)
        idxm = jnp.where(lane == k, am, idxm)
        vals = jnp.where(col == am, USEDD, vals)
        return vals, idxm

    _, idxm = lax.fori_loop(0, K, step, (vals0, idx0))
    return idxm


def _topk_idx(x_ref, bc_ref, br_ref, i, idx_ref):
    xall = x_ref[...]
    xrb = x_ref[pl.ds(i * RB, RB), :]
    g = jax.lax.dot_general(xrb, xall, (((1,), (1,)), ((), ())),
                            preferred_element_type=jnp.float32)
    x2 = xall * xall
    ones_row = jnp.ones((1, D), jnp.float32)
    d2row = jax.lax.dot_general(ones_row, x2, (((1,), (1,)), ((), ())),
                                preferred_element_type=jnp.float32,
                                precision=lax.Precision.HIGHEST)
    d2col = jnp.sum(xrb * xrb, axis=1, keepdims=True)
    dist = d2col - 2.0 * g + d2row

    bad = bc_ref[...] != br_ref[...]
    vals0 = jnp.where(bad, BIGD, dist)

    l128 = lax.broadcasted_iota(jnp.int32, (RB, 128), 1)
    mq = [jnp.full((RB, 128), SENT, jnp.float32) for _ in range(Q)]
    jq = [jnp.zeros((RB, 128), jnp.int32) for _ in range(Q)]
    for c in range(NCH):
        v = vals0[:, c * 128 : (c + 1) * 128]
        jfill = l128 + (c * 128)
        b = [v < mq[t] for t in range(Q)]
        for t in range(Q - 1, 0, -1):
            mq[t] = jnp.where(b[t], jnp.where(b[t - 1], mq[t - 1], v), mq[t])
            jq[t] = jnp.where(b[t], jnp.where(b[t - 1], jq[t - 1], jfill), jq[t])
        mq[0] = jnp.where(b[0], v, mq[0])
        jq[0] = jnp.where(b[0], jfill, jq[0])

    lane = lax.broadcasted_iota(jnp.int32, (RB, K), 1)
    idxm = jnp.zeros((RB, K), jnp.int32)
    pops = jnp.zeros((RB, 128), jnp.int32)

    def pstep(k, carry):
        mq, jq, idxm, pops = carry
        mq = list(mq)
        jq = list(jq)
        m = jnp.min(mq[0], axis=1, keepdims=True)
        jc = jnp.where(mq[0] == m, jq[0], BIGI)
        am = jnp.min(jc, axis=1, keepdims=True)
        idxm = jnp.where(lane == k, am, idxm)
        popm = jc == am
        for t in range(Q - 1):
            mq[t] = jnp.where(popm, mq[t + 1], mq[t])
            jq[t] = jnp.where(popm, jq[t + 1], jq[t])
        mq[Q - 1] = jnp.where(popm, SENT, mq[Q - 1])
        pops = pops + popm.astype(jnp.int32)
        return tuple(mq), tuple(jq), idxm, pops

    (_, _, idxm, pops) = lax.fori_loop(
        0, K, pstep, (tuple(mq), tuple(jq), idxm, pops))
    idx_ref[...] = idxm

    overflow = jnp.any(pops >= Q)

    @pl.when(overflow)
    def _():
        idx_ref[...] = _topk_slow(vals0)


def _l1_body(x_ref, bc_ref, br_ref, w_ref, idx_ref, a_ref):
    i = pl.program_id(0)
    _topk_idx(x_ref, bc_ref, br_ref, i, idx_ref)
    xrb = x_ref[pl.ds(i * RB, RB), :]
    wt = w_ref[0:D, :]
    a_ref[...] = jax.lax.dot_general(xrb, wt, (((1,), (0,)), ((), ())),
                                     preferred_element_type=jnp.float32)


def _l1_call(x, bcol, brow, W):
    return pl.pallas_call(
        _l1_body,
        grid=(NRB,),
        in_specs=[
            pl.BlockSpec((N, D), lambda i: (0, 0)),
            pl.BlockSpec((RB, 1), lambda i: (i, 0)),
            pl.BlockSpec((1, N), lambda i: (0, 0)),
            pl.BlockSpec((2 * D, D), lambda i: (0, 0)),
        ],
        out_specs=[
            pl.BlockSpec((RB, K), lambda i: (i, 0)),
            pl.BlockSpec((RB, D), lambda i: (i, 0)),
        ],
        out_shape=[
            jax.ShapeDtypeStruct((N, K), jnp.int32),
            jax.ShapeDtypeStruct((N, D), jnp.float32),
        ],
    )(x, bcol, brow, W)


def _l2_body(x_ref, bc_ref, br_ref, w_ref, b_ref, idx_ref, a_ref, bm_ref):
    i = pl.program_id(0)
    _topk_idx(x_ref, bc_ref, br_ref, i, idx_ref)
    xrb = x_ref[pl.ds(i * RB, RB), :]
    wt = w_ref[0:D, :]
    wb = w_ref[D : 2 * D, :]
    a_ref[...] = (
        jax.lax.dot_general(xrb, wt - wb, (((1,), (0,)), ((), ())),
                            preferred_element_type=jnp.float32,
                            precision=lax.Precision.HIGHEST)
        + b_ref[...]
    )
    bm_ref[...] = jax.lax.dot_general(xrb, wb, (((1,), (0,)), ((), ())),
                                      preferred_element_type=jnp.float32,
                                      precision=lax.Precision.HIGHEST)


def _l2_call(x, bcol, brow, W, brow_bias):
    return pl.pallas_call(
        _l2_body,
        grid=(NRB,),
        in_specs=[
            pl.BlockSpec((N, D), lambda i: (0, 0)),
            pl.BlockSpec((RB, 1), lambda i: (i, 0)),
            pl.BlockSpec((1, N), lambda i: (0, 0)),
            pl.BlockSpec((2 * D, D), lambda i: (0, 0)),
            pl.BlockSpec((1, D), lambda i: (0, 0)),
        ],
        out_specs=[
            pl.BlockSpec((RB, K), lambda i: (i, 0)),
            pl.BlockSpec((RB, D), lambda i: (i, 0)),
            pl.BlockSpec((RB, D), lambda i: (i, 0)),
        ],
        out_shape=[
            jax.ShapeDtypeStruct((N, K), jnp.int32),
            jax.ShapeDtypeStruct((N, D), jnp.float32),
            jax.ShapeDtypeStruct((N, D), jnp.float32),
        ],
    )(x, bcol, brow, W, brow_bias)


_NC = 2
_NS = 16
_L = 16
_NW = _NC * _NS
_RPW = N // _NW


def _mesh():
    return plsc.VectorSubcoreMesh(core_axis_name="c", subcore_axis_name="s",
                                  num_cores=_NC, num_subcores=_NS)


def _wid_base():
    wid = lax.axis_index("s") * _NC + lax.axis_index("c")
    return wid * _RPW


_BR = 4
_NBAT = _RPW // _BR


def _sc_gather_body(idx_hbm, x_hbm, out_hbm, idx_v, r0, r1, sem0, sem1):
    base = _wid_base()
    pltpu.sync_copy(idx_hbm.at[pl.ds(base * K, _RPW * K)], idx_v)

    def start(u, buf, sem):
        pltpu.async_copy(x_hbm.at[idx_v.at[pl.ds(u * (_BR * K), _BR * K)]],
                         buf, sem)

    def wait(buf, sem):
        pltpu.make_async_copy(x_hbm.at[pl.ds(0, _BR * K)], buf, sem).wait()

    def flush(u, buf):
        pltpu.sync_copy(buf, out_hbm.at[pl.ds((base + u * _BR) * K, _BR * K)])

    start(0, r0, sem0)

    def body2(b2, carry):
        u = 2 * b2
        start(u + 1, r1, sem1)
        wait(r0, sem0)
        flush(u, r0)

        @pl.when(b2 < _NBAT // 2 - 1)
        def _():
            start(u + 2, r0, sem0)

        wait(r1, sem1)
        flush(u + 1, r1)
        return carry

    lax.fori_loop(0, _NBAT // 2, body2, 0)


@functools.cache
def _sc_gather_call():
    return pl.kernel(
        _sc_gather_body,
        out_type=jax.ShapeDtypeStruct((N * K, D), jnp.float32),
        mesh=_mesh(),
        scratch_types=[
            pltpu.VMEM((_RPW * K,), jnp.int32),
            pltpu.VMEM((_BR * K, D), jnp.float32),
            pltpu.VMEM((_BR * K, D), jnp.float32),
            pltpu.SemaphoreType.DMA,
            pltpu.SemaphoreType.DMA,
        ],
    )


def _sc_gather_max_body(idx_hbm, bm_hbm, a_hbm, out_hbm,
                        idx_v, a_v, hb, r0, r1, sem0, sem1):
    base = _wid_base()
    pltpu.sync_copy(idx_hbm.at[pl.ds(base * K, _RPW * K)], idx_v)
    pltpu.sync_copy(a_hbm.at[pl.ds(base, _RPW)], a_v)

    def start(u, buf, sem):
        pltpu.async_copy(bm_hbm.at[idx_v.at[pl.ds(u * (_BR * K), _BR * K)]],
                         buf, sem)

    def wait(buf, sem):
        pltpu.make_async_copy(bm_hbm.at[pl.ds(0, _BR * K)], buf, sem).wait()

    def compute(u, buf):
        def rowbody(rr, carry):
            for g in range(D // _L):
                sl = pl.ds(g * _L, _L)
                acc = buf[rr * K, sl]
                for j in range(1, K):
                    acc = jnp.maximum(acc, buf[rr * K + j, sl])
                hb[rr, sl] = jnp.maximum(acc + a_v[u * _BR + rr, sl], 0.0)
            return carry

        lax.fori_loop(0, _BR, rowbody, 0)
        pltpu.sync_copy(hb, out_hbm.at[pl.ds(base + u * _BR, _BR)])

    start(0, r0, sem0)

    def body2(b2, carry):
        u = 2 * b2
        start(u + 1, r1, sem1)
        wait(r0, sem0)
        compute(u, r0)

        @pl.when(b2 < _NBAT // 2 - 1)
        def _():
            start(u + 2, r0, sem0)

        wait(r1, sem1)
        compute(u + 1, r1)
        return carry

    lax.fori_loop(0, _NBAT // 2, body2, 0)


@functools.cache
def _sc_gather_max_call():
    return pl.kernel(
        _sc_gather_max_body,
        out_type=jax.ShapeDtypeStruct((N, D), jnp.float32),
        mesh=_mesh(),
        scratch_types=[
            pltpu.VMEM((_RPW * K,), jnp.int32),
            pltpu.VMEM((_RPW, D), jnp.float32),
            pltpu.VMEM((_BR, D), jnp.float32),
            pltpu.VMEM((_BR * K, D), jnp.float32),
            pltpu.VMEM((_BR * K, D), jnp.float32),
            pltpu.SemaphoreType.DMA,
            pltpu.SemaphoreType.DMA,
        ],
    )


def _edge_body(xj_ref, x_ref, w_ref, b_ref, a_ref, h_ref):
    xj3 = xj_ref[...].reshape(EB, K, D)
    xi = x_ref[...]
    diff = (xj3 - xi[:, None, :]).astype(jnp.bfloat16).reshape(EB * K, D)
    wb = w_ref[D : 2 * D, :].astype(jnp.bfloat16)
    e = jax.lax.dot_general(diff, wb, (((1,), (0,)), ((), ())),
                            preferred_element_type=jnp.float32)
    p = e.reshape(EB, K, D) + a_ref[...][:, None, :] + b_ref[...][None, :, :]
    h_ref[...] = jnp.max(jnp.maximum(p, 0.0), axis=1)


def _edge_call(xj, x, W, b, a):
    return pl.pallas_call(
        _edge_body,
        grid=(NEB,),
        in_specs=[
            pl.BlockSpec((EB * K, D), lambda i: (i, 0)),
            pl.BlockSpec((EB, D), lambda i: (i, 0)),
            pl.BlockSpec((2 * D, D), lambda i: (0, 0)),
            pl.BlockSpec((1, D), lambda i: (0, 0)),
            pl.BlockSpec((EB, D), lambda i: (i, 0)),
        ],
        out_specs=pl.BlockSpec((EB, D), lambda i: (i, 0)),
        out_shape=jax.ShapeDtypeStruct((N, D), jnp.float32),
    )(xj, x, W, b, a)


def _loss_body(h_ref, y_ref, o_ref):
    i = pl.program_id(0)

    @pl.when(i == 0)
    def _():
        o_ref[...] = jnp.zeros_like(o_ref)

    d = h_ref[...] - y_ref[...]
    o_ref[...] += jnp.sum(d * d).reshape(1, 1) / jnp.float32(N * D)


def _tc_loss(h, y):
    return pl.pallas_call(
        _loss_body,
        grid=(NRB,),
        in_specs=[
            pl.BlockSpec((RB, D), lambda i: (i, 0)),
            pl.BlockSpec((RB, D), lambda i: (i, 0)),
        ],
        out_specs=pl.BlockSpec((1, 1), lambda i: (0, 0)),
        out_shape=jax.ShapeDtypeStruct((1, 1), jnp.float32),
    )(h, y)


def kernel(x, batch, y, W0, b0, W1, b1):
    bf = batch.astype(jnp.float32)
    bcol = bf.reshape(N, 1)
    brow = bf.reshape(1, N)

    idx0, a1 = _l1_call(x, bcol, brow, W0)
    xj = _sc_gather_call()(idx0.reshape(N * K), x)
    h1 = _edge_call(xj, x, W0, b0.reshape(1, D), a1)

    idx1, a2, bm2 = _l2_call(h1, bcol, brow, W1, b1.reshape(1, D))
    h2 = _sc_gather_max_call()(idx1.reshape(N * K), bm2, a2)

    loss = _tc_loss(h2, y)[0, 0]
    return (h2, loss)
